# baseline (speedup 1.0000x reference)
import functools

import numpy as np
import jax
import jax.numpy as jnp
from jax import lax
from jax.experimental import pallas as pl
from jax.experimental.pallas import tpu as pltpu
from jax.experimental.pallas import tpu_sc as plsc

_B = 16384
_EMB = 64
_NC, _NS = 2, 16
_NW = _NC * _NS
_BPW = _B // _NW
_CH = 128
_NCHUNK = _BPW // _CH

_TILE = 2048


def _sc_gather_body(u_tab, c_tab, uids, cids, u_out, c_out,
                    idx_u, idx_c, rows_u, rows_c, sem):
    wid = lax.axis_index("s") * _NC + lax.axis_index("c")
    base = wid * _BPW
    pltpu.sync_copy(uids.at[wid], idx_u)
    pltpu.sync_copy(cids.at[wid], idx_c)
    copies = []
    for j in range(_NCHUNK):
        copies.append(pltpu.async_copy(
            u_tab.at[idx_u.at[j]], rows_u.at[pl.ds(j * _CH, _CH)], sem))
        copies.append(pltpu.async_copy(
            c_tab.at[idx_c.at[j]], rows_c.at[pl.ds(j * _CH, _CH)], sem))
    for cp in copies:
        cp.wait()
    pltpu.sync_copy(rows_u, u_out.at[pl.ds(base, _BPW)])
    pltpu.sync_copy(rows_c, c_out.at[pl.ds(base, _BPW)])


@functools.cache
def _sc_gather():
    return pl.kernel(
        _sc_gather_body,
        out_type=(jax.ShapeDtypeStruct((_B, _EMB), jnp.float32),
                  jax.ShapeDtypeStruct((_B, _EMB), jnp.float32)),
        mesh=plsc.VectorSubcoreMesh(core_axis_name="c", subcore_axis_name="s"),
        compiler_params=pltpu.CompilerParams(use_tc_tiling_on_sc=False),
        scratch_types=[
            pltpu.VMEM((_NCHUNK, _CH), jnp.int32),
            pltpu.VMEM((_NCHUNK, _CH), jnp.int32),
            pltpu.VMEM((_BPW, _EMB), jnp.float32),
            pltpu.VMEM((_BPW, _EMB), jnp.float32),
            pltpu.SemaphoreType.DMA,
        ],
    )


def _mlp_body(u_ref, c_ref, w0u, w0c, w1, w2, w3,
              b0, b1, b2, b3, g0, g1, g2, be0, be1, be2, out_ref):
    s = np.float32(1.0 / np.sqrt(1.0 + 1e-5))

    def dot_t(x, w):
        return lax.dot_general(x, w, (((1,), (1,)), ((), ())),
                               preferred_element_type=jnp.float32)

    h = dot_t(u_ref[...], w0u[...]) + dot_t(c_ref[...], w0c[...])
    h = jnp.maximum(h + b0[...], 0.0) * (g0[...] * s) + be0[...]
    h = dot_t(h, w1[...])
    h = jnp.maximum(h + b1[...], 0.0) * (g1[...] * s) + be1[...]
    h = dot_t(h, w2[...])
    h = jnp.maximum(h + b2[...], 0.0) * (g2[...] * s) + be2[...]
    h = dot_t(h, w3[...]) + b3[...]
    out_ref[...] = jax.nn.sigmoid(h)


def _mlp_call(u_emb, c_emb, wb):
    full = lambda shape: pl.BlockSpec(shape, lambda i: (0, 0))
    return pl.pallas_call(
        _mlp_body,
        grid=(_B // _TILE,),
        in_specs=[
            pl.BlockSpec((_TILE, _EMB), lambda i: (i, 0)),
            pl.BlockSpec((_TILE, _EMB), lambda i: (i, 0)),
        ] + [full(w.shape) for w in wb],
        out_specs=pl.BlockSpec((_TILE, 8), lambda i: (i, 0)),
        out_shape=jax.ShapeDtypeStruct((_B, 8), jnp.float32),
    )(u_emb, c_emb, *wb)


def kernel(user_ids, course_ids, params):
    uids = jnp.asarray(user_ids, jnp.int32).reshape(_NW, _NCHUNK, _CH)
    cids = jnp.asarray(course_ids, jnp.int32).reshape(_NW, _NCHUNK, _CH)
    u_emb, c_emb = _sc_gather()(params['user_table'], params['course_table'],
                                uids, cids)
    p = params
    row = lambda v: v.reshape(1, -1)
    w3p = jnp.pad(p['W3'], ((0, 7), (0, 0)))
    b3p = jnp.pad(p['b3'], (0, 7)).reshape(1, 8)
    wb = (p['W0'][:, :_EMB], p['W0'][:, _EMB:], p['W1'], p['W2'], w3p,
          row(p['b0']), row(p['b1']), row(p['b2']), b3p,
          row(p['gamma0']), row(p['gamma1']), row(p['gamma2']),
          row(p['beta0']), row(p['beta1']), row(p['beta2']))
    out = _mlp_call(u_emb, c_emb, wb)
    return out[:, 0]
